# Initial kernel scaffold; baseline (speedup 1.0000x reference)
import jax
import jax.numpy as jnp
from jax.experimental import pallas as pl


def kernel(q, k, v):
    raise NotImplementedError("write your pallas kernel here")


# speedup vs baseline: 1.4263x; 1.4263x over previous
import math

import jax
import jax.numpy as jnp
from jax.experimental import pallas as pl
from jax.experimental.pallas import tpu as pltpu

_B, _S, _H, _DH = 2, 2048, 16, 128
_BQ = 1024
_SCALE = math.sqrt(_DH)


def _dot_nt(a, b):
    return jax.lax.dot_general(
        a, b, (((1,), (1,)), ((), ())),
        preferred_element_type=jnp.float32,
    )


def _fused_attn_body(qh_ref, ql_ref, kh_ref, kl_ref, v_ref,
                     score_ref, out_ref):
    qh, ql = qh_ref[0], ql_ref[0]
    kh, kl = kh_ref[0], kl_ref[0]
    x = _dot_nt(qh, kh) + _dot_nt(qh, kl) + _dot_nt(ql, kh)
    score_ref[0, 0] = jnp.abs(x)
    m = jnp.max(x, axis=1, keepdims=True)
    p = jnp.exp(x - m)
    s = jnp.sum(p, axis=1, keepdims=True)
    o = jax.lax.dot_general(
        p, v_ref[0], (((1,), (0,)), ((), ())),
        precision=jax.lax.Precision.DEFAULT,
        preferred_element_type=jnp.float32,
    )
    out_ref[0] = o / s


def _split_hi_lo(x):
    hi = x.astype(jnp.bfloat16)
    lo = (x - hi.astype(jnp.float32)).astype(jnp.bfloat16)
    return hi, lo


def kernel(q, k, v):
    qs = q.reshape(_B, _S, _H * _DH) * _SCALE
    kf = k.reshape(_B, _S, _H * _DH)
    vf = v.reshape(_B, _S, _H * _DH)
    q_hi, q_lo = _split_hi_lo(qs)
    k_hi, k_lo = _split_hi_lo(kf)
    grid = (_B, _H, _S // _BQ)
    q_spec = pl.BlockSpec((1, _BQ, _DH), lambda b, h, i: (b, i, h))
    kv_spec = pl.BlockSpec((1, _S, _DH), lambda b, h, i: (b, 0, h))
    score, out = pl.pallas_call(
        _fused_attn_body,
        grid=grid,
        in_specs=[q_spec, q_spec, kv_spec, kv_spec, kv_spec],
        out_specs=[
            pl.BlockSpec((1, 1, _BQ, _S), lambda b, h, i: (b, h, i, 0)),
            pl.BlockSpec((1, _BQ, _DH), lambda b, h, i: (b, i, h)),
        ],
        out_shape=[
            jax.ShapeDtypeStruct((_B, _H, _S, _S), jnp.float32),
            jax.ShapeDtypeStruct((_B, _S, _H * _DH), jnp.float32),
        ],
        compiler_params=pltpu.CompilerParams(
            dimension_semantics=("parallel", "parallel", "arbitrary"),
        ),
    )(q_hi, q_lo, k_hi, k_lo, vf)
    return out, score


# speedup vs baseline: 1.4394x; 1.0092x over previous
import math

import jax
import jax.numpy as jnp
from jax.experimental import pallas as pl
from jax.experimental.pallas import tpu as pltpu

_B, _S, _H, _DH = 2, 2048, 16, 128
_BQ = 2048
_SCALE = math.sqrt(_DH)


def _dot_nt(a, b):
    return jax.lax.dot_general(
        a, b, (((1,), (1,)), ((), ())),
        preferred_element_type=jnp.float32,
    )


def _fused_attn_body(qh_ref, ql_ref, kh_ref, kl_ref, v_ref,
                     score_ref, out_ref):
    qh, ql = qh_ref[0], ql_ref[0]
    kh, kl = kh_ref[0], kl_ref[0]
    x = _dot_nt(qh, kh) + _dot_nt(qh, kl) + _dot_nt(ql, kh)
    score_ref[0, 0] = jnp.abs(x)
    m = jnp.max(x, axis=1, keepdims=True)
    p = jnp.exp(x - m)
    s = jnp.sum(p, axis=1, keepdims=True)
    o = jax.lax.dot_general(
        p, v_ref[0], (((1,), (0,)), ((), ())),
        precision=jax.lax.Precision.DEFAULT,
        preferred_element_type=jnp.float32,
    )
    out_ref[0] = o / s


def _split_hi_lo(x):
    hi = x.astype(jnp.bfloat16)
    lo = (x - hi.astype(jnp.float32)).astype(jnp.bfloat16)
    return hi, lo


def kernel(q, k, v):
    qs = q.reshape(_B, _S, _H * _DH) * _SCALE
    kf = k.reshape(_B, _S, _H * _DH)
    vf = v.reshape(_B, _S, _H * _DH)
    q_hi, q_lo = _split_hi_lo(qs)
    k_hi, k_lo = _split_hi_lo(kf)
    grid = (_B, _H, _S // _BQ)
    q_spec = pl.BlockSpec((1, _BQ, _DH), lambda b, h, i: (b, i, h))
    kv_spec = pl.BlockSpec((1, _S, _DH), lambda b, h, i: (b, 0, h))
    score, out = pl.pallas_call(
        _fused_attn_body,
        grid=grid,
        in_specs=[q_spec, q_spec, kv_spec, kv_spec, kv_spec],
        out_specs=[
            pl.BlockSpec((1, 1, _BQ, _S), lambda b, h, i: (b, h, i, 0)),
            pl.BlockSpec((1, _BQ, _DH), lambda b, h, i: (b, i, h)),
        ],
        out_shape=[
            jax.ShapeDtypeStruct((_B, _H, _S, _S), jnp.float32),
            jax.ShapeDtypeStruct((_B, _S, _H * _DH), jnp.float32),
        ],
        compiler_params=pltpu.CompilerParams(
            dimension_semantics=("parallel", "parallel", "arbitrary"),
        ),
    )(q_hi, q_lo, k_hi, k_lo, vf)
    return out, score
